# baseline (speedup 1.0000x reference)
import functools

import jax
import jax.numpy as jnp
from jax import lax
from jax.experimental import pallas as pl
from jax.experimental.pallas import tpu as pltpu
from jax.experimental.pallas import tpu_sc as plsc

_VOCAB = 8192
_EMBED = 256
_ROWS = 8192
_TM = 512
_RT = _ROWS // _TM
_W = 256


def _argmin_body(z_ref, zn_ref, cn_ref, colf_ref, cb_ref,
                 idx_ref, ppl_ref, loss_ref, scr_ref, acc_ref):
    r = pl.program_id(0)
    zt = z_ref[0] * -2.0
    zn = zn_ref[...]

    val = None
    for k in range(_VOCAB // _W):
        off = k * _W
        cbk = cb_ref[off:off + _W, :]
        mmk = lax.dot_general(zt, cbk, (((0,), (1,)), ((), ())),
                              preferred_element_type=jnp.float32)
        cnk = cn_ref[:, pl.ds(off, _W)]
        colk = colf_ref[:, pl.ds(off, _W)]
        dc = (zn + cnk) + mmk
        dk0, dk1 = dc[:, :128], dc[:, 128:]
        ck0, ck1 = colk[:, :128], colk[:, 128:]
        cf = dk1 < dk0
        dk = jnp.where(cf, dk1, dk0)
        ik = jnp.where(cf, jnp.broadcast_to(ck1, (_TM, 128)),
                       jnp.broadcast_to(ck0, (_TM, 128)))
        if val is None:
            val, idx = dk, ik
        else:
            c = dk < val
            idx = jnp.where(c, ik, idx)
            val = jnp.where(c, dk, val)

    m = jnp.min(val, axis=1, keepdims=True)
    lif = jnp.min(jnp.where(val == m, idx, jnp.inf), axis=1, keepdims=True)
    idx_ref[...] = lif.astype(jnp.int32)
    scr_ref[pl.ds(r, 1)] = lif.reshape(1, 1, _TM)
    part = jnp.sum(m)

    @pl.when(r == 0)
    def _init():
        acc_ref[0, 0] = part

    @pl.when(r > 0)
    def _acc():
        acc_ref[0, 0] = acc_ref[0, 0] + part

    @pl.when(r == _RT - 1)
    def _finish():
        loss_ref[...] = jnp.full(
            (1, 1), acc_ref[0, 0] / float(_ROWS * _EMBED), jnp.float32)
        ent = jnp.float32(0.0)
        for h in range(2):
            vals = [scr_ref[2 * b + h] for b in range(8)]
            for b in range(8):
                cnt = jnp.zeros((1, _TM), jnp.float32)
                for b2 in range(8):
                    cnt = cnt + (vals[b] == vals[b2]).astype(jnp.float32)
                ent = ent + jnp.sum(jnp.log(cnt * 0.125 + 1e-10))
        ppl_ref[...] = jnp.full((1, 1), jnp.exp(-(ent * 0.125)), jnp.float32)


def _run_argmin(z3, zn, cn2, cb):
    colf = jnp.arange(_VOCAB, dtype=jnp.float32).reshape(1, _VOCAB)
    return pl.pallas_call(
        _argmin_body,
        grid=(_RT,),
        in_specs=[
            pl.BlockSpec((1, _EMBED, _TM), lambda r: (r // 2, 0, r % 2)),
            pl.BlockSpec((_TM, 1), lambda r: (r, 0)),
            pl.BlockSpec((1, _VOCAB), lambda r: (0, 0)),
            pl.BlockSpec((1, _VOCAB), lambda r: (0, 0)),
            pl.BlockSpec((_VOCAB, _EMBED), lambda r: (0, 0)),
        ],
        out_specs=[
            pl.BlockSpec((_TM, 1), lambda r: (r, 0)),
            pl.BlockSpec((1, 1), lambda r: (0, 0)),
            pl.BlockSpec((1, 1), lambda r: (0, 0)),
        ],
        out_shape=[
            jax.ShapeDtypeStruct((_ROWS, 1), jnp.int32),
            jax.ShapeDtypeStruct((1, 1), jnp.float32),
            jax.ShapeDtypeStruct((1, 1), jnp.float32),
        ],
        scratch_shapes=[
            pltpu.VMEM((_RT, 1, _TM), jnp.float32),
            pltpu.SMEM((1, 1), jnp.float32),
        ],
    )(z3, zn, cn2, colf, cb)


_NC = 2
_NS = 16
_NW = _NC * _NS
_BPW = _ROWS // _NW
_CH = 128
_NCH = _BPW // _CH


@functools.cache
def _make_sc_gather():
    mesh = plsc.VectorSubcoreMesh(core_axis_name="c", subcore_axis_name="s")

    @functools.partial(
        pl.kernel,
        mesh=mesh,
        out_type=jax.ShapeDtypeStruct((_ROWS, _EMBED), jnp.float32),
        scratch_types=[
            pltpu.VMEM((_CH,), jnp.int32),
            pltpu.VMEM((_CH,), jnp.int32),
            pltpu.VMEM((_CH, _EMBED), jnp.float32),
            pltpu.VMEM((_CH, _EMBED), jnp.float32),
            pltpu.SemaphoreType.DMA,
            pltpu.SemaphoreType.DMA,
        ],
    )
    def _sc_gather(table_hbm, idx_hbm, out_hbm, idx0, idx1, rows0, rows1, sem0, sem1):
        wid = lax.axis_index("s") * _NC + lax.axis_index("c")
        base = wid * _BPW
        idx_bufs = (idx0, idx1)
        row_bufs = (rows0, rows1)
        sems = (sem0, sem1)
        copies = []
        for k in range(_NCH):
            pltpu.sync_copy(idx_hbm.at[pl.ds(base + k * _CH, _CH)], idx_bufs[k])
            copies.append(pltpu.async_copy(table_hbm.at[idx_bufs[k]], row_bufs[k], sems[k]))
        for k in range(_NCH):
            copies[k].wait()
            pltpu.sync_copy(row_bufs[k], out_hbm.at[pl.ds(base + k * _CH, _CH)])

    return _sc_gather


def kernel(z, codebook):
    B, C, H, W = z.shape
    z_flat = jnp.transpose(z, (0, 2, 3, 1)).reshape(B, H * W, C).astype(jnp.float32)
    cb = codebook.astype(jnp.float32)
    zn = jnp.sum(z_flat ** 2, axis=-1, keepdims=True)
    cn = jnp.sum(cb ** 2, axis=-1)

    idx2, ppl_out, loss_out = _run_argmin(
        z.reshape(B, C, H * W), zn.reshape(B * H * W, 1),
        cn.reshape(1, _VOCAB), cb)

    idx_flat = idx2.reshape(B * H * W)
    zq_flat = _make_sc_gather()(cb, idx_flat)

    z_q = jnp.transpose(zq_flat.reshape(B, H, W, C), (0, 3, 1, 2))
    indices = idx_flat.reshape(B, H, W)
    return z_q, indices, loss_out[0, 0], ppl_out[0, 0]


# speedup vs baseline: 1.0422x; 1.0422x over previous
import functools

import jax
import jax.numpy as jnp
from jax import lax
from jax.experimental import pallas as pl
from jax.experimental.pallas import tpu as pltpu
from jax.experimental.pallas import tpu_sc as plsc

_VOCAB = 8192
_EMBED = 256
_ROWS = 8192
_TM = 512
_RT = _ROWS // _TM
_W = 256


def _argmin_body(z_ref, zn_ref, cn_ref, colf_ref, cb_ref,
                 idx_ref, loss_ref, acc_ref):
    r = pl.program_id(0)
    zt = z_ref[0] * -2.0
    zn = zn_ref[...]

    val = None
    for k in range(_VOCAB // _W):
        off = k * _W
        cbk = cb_ref[off:off + _W, :]
        mmk = lax.dot_general(zt, cbk, (((0,), (1,)), ((), ())),
                              preferred_element_type=jnp.float32)
        cnk = cn_ref[:, pl.ds(off, _W)]
        colk = colf_ref[:, pl.ds(off, _W)]
        dc = (zn + cnk) + mmk
        dk0, dk1 = dc[:, :128], dc[:, 128:]
        ck0, ck1 = colk[:, :128], colk[:, 128:]
        cf = dk1 < dk0
        dk = jnp.where(cf, dk1, dk0)
        ik = jnp.where(cf, jnp.broadcast_to(ck1, (_TM, 128)),
                       jnp.broadcast_to(ck0, (_TM, 128)))
        if val is None:
            val, idx = dk, ik
        else:
            c = dk < val
            idx = jnp.where(c, ik, idx)
            val = jnp.where(c, dk, val)

    m = jnp.min(val, axis=1, keepdims=True)
    lif = jnp.min(jnp.where(val == m, idx, jnp.inf), axis=1, keepdims=True)
    idx_ref[...] = lif.astype(jnp.int32)
    part = jnp.sum(m)

    @pl.when(r == 0)
    def _init():
        acc_ref[0, 0] = part

    @pl.when(r > 0)
    def _acc():
        acc_ref[0, 0] = acc_ref[0, 0] + part

    @pl.when(r == _RT - 1)
    def _finish():
        loss_ref[...] = jnp.full(
            (1, 1), acc_ref[0, 0] / float(_ROWS * _EMBED), jnp.float32)


def _run_argmin(z3, zn, cn2, cb):
    colf = jnp.arange(_VOCAB, dtype=jnp.float32).reshape(1, _VOCAB)
    return pl.pallas_call(
        _argmin_body,
        grid=(_RT,),
        in_specs=[
            pl.BlockSpec((1, _EMBED, _TM), lambda r: (r // 2, 0, r % 2)),
            pl.BlockSpec((_TM, 1), lambda r: (r, 0)),
            pl.BlockSpec((1, _VOCAB), lambda r: (0, 0)),
            pl.BlockSpec((1, _VOCAB), lambda r: (0, 0)),
            pl.BlockSpec((_VOCAB, _EMBED), lambda r: (0, 0)),
        ],
        out_specs=[
            pl.BlockSpec((_TM, 1), lambda r: (r, 0)),
            pl.BlockSpec((1, 1), lambda r: (0, 0)),
        ],
        out_shape=[
            jax.ShapeDtypeStruct((_ROWS, 1), jnp.int32),
            jax.ShapeDtypeStruct((1, 1), jnp.float32),
        ],
        scratch_shapes=[
            pltpu.SMEM((1, 1), jnp.float32),
        ],
    )(z3, zn, cn2, colf, cb)


def _ppl_body(idx_ref, out_ref):
    idx = idx_ref[...]
    c = jnp.zeros(idx.shape, jnp.int32)
    for b in range(8):
        c = c + (idx == idx[b:b + 1, :]).astype(jnp.int32)
    p = c.astype(jnp.float32) * 0.125
    ent = jnp.sum(jnp.log(p + 1e-10)) * 0.125
    out_ref[...] = jnp.full((1, 1), jnp.exp(-ent), jnp.float32)


def _run_ppl(idx8):
    return pl.pallas_call(
        _ppl_body,
        out_shape=jax.ShapeDtypeStruct((1, 1), jnp.float32),
    )(idx8)


_NC = 2
_NS = 16
_NW = _NC * _NS
_BPW = _ROWS // _NW
_CH = 128
_NCH = _BPW // _CH


@functools.cache
def _make_sc_gather():
    mesh = plsc.VectorSubcoreMesh(core_axis_name="c", subcore_axis_name="s")

    @functools.partial(
        pl.kernel,
        mesh=mesh,
        out_type=jax.ShapeDtypeStruct((_ROWS, _EMBED), jnp.float32),
        scratch_types=[
            pltpu.VMEM((_CH,), jnp.int32),
            pltpu.VMEM((_CH,), jnp.int32),
            pltpu.VMEM((_CH, _EMBED), jnp.float32),
            pltpu.VMEM((_CH, _EMBED), jnp.float32),
            pltpu.SemaphoreType.DMA,
            pltpu.SemaphoreType.DMA,
        ],
    )
    def _sc_gather(table_hbm, idx_hbm, out_hbm, idx0, idx1, rows0, rows1, sem0, sem1):
        wid = lax.axis_index("s") * _NC + lax.axis_index("c")
        base = wid * _BPW
        idx_bufs = (idx0, idx1)
        row_bufs = (rows0, rows1)
        sems = (sem0, sem1)
        copies = []
        for k in range(_NCH):
            pltpu.sync_copy(idx_hbm.at[pl.ds(base + k * _CH, _CH)], idx_bufs[k])
            copies.append(pltpu.async_copy(table_hbm.at[idx_bufs[k]], row_bufs[k], sems[k]))
        for k in range(_NCH):
            copies[k].wait()
            pltpu.sync_copy(row_bufs[k], out_hbm.at[pl.ds(base + k * _CH, _CH)])

    return _sc_gather


def kernel(z, codebook):
    B, C, H, W = z.shape
    z_flat = jnp.transpose(z, (0, 2, 3, 1)).reshape(B, H * W, C).astype(jnp.float32)
    cb = codebook.astype(jnp.float32)
    zn = jnp.sum(z_flat ** 2, axis=-1, keepdims=True)
    cn = jnp.sum(cb ** 2, axis=-1)

    idx2, loss_out = _run_argmin(
        z.reshape(B, C, H * W), zn.reshape(B * H * W, 1),
        cn.reshape(1, _VOCAB), cb)

    idx_flat = idx2.reshape(B * H * W)
    zq_flat = _make_sc_gather()(cb, idx_flat)
    ppl_out = _run_ppl(idx2.reshape(B, H * W))

    z_q = jnp.transpose(zq_flat.reshape(B, H, W, C), (0, 3, 1, 2))
    indices = idx_flat.reshape(B, H, W)
    return z_q, indices, loss_out[0, 0], ppl_out[0, 0]


# speedup vs baseline: 1.0511x; 1.0085x over previous
import functools

import jax
import jax.numpy as jnp
from jax import lax
from jax.experimental import pallas as pl
from jax.experimental.pallas import tpu as pltpu
from jax.experimental.pallas import tpu_sc as plsc

_VOCAB = 8192
_EMBED = 256
_ROWS = 8192
_TM = 512
_RT = _ROWS // _TM
_W = 512


def _argmin_body(z_ref, zn_ref, cn_ref, colf_ref, cb_ref,
                 idx_ref, loss_ref, acc_ref):
    r = pl.program_id(0)
    zt = z_ref[0] * -2.0
    zn = zn_ref[...]

    val = None
    for k in range(_VOCAB // _W):
        off = k * _W
        cbk = cb_ref[off:off + _W, :]
        mmk = lax.dot_general(zt, cbk, (((0,), (1,)), ((), ())),
                              preferred_element_type=jnp.float32)
        cnk = cn_ref[:, pl.ds(off, _W)]
        colk = colf_ref[:, pl.ds(off, _W)]
        dc = (zn + cnk) + mmk
        dk = None
        for q in range(_W // 128):
            dq = dc[:, q * 128:(q + 1) * 128]
            cq = jnp.broadcast_to(colk[:, q * 128:(q + 1) * 128], (_TM, 128))
            if dk is None:
                dk, ik = dq, cq
            else:
                cf = dq < dk
                ik = jnp.where(cf, cq, ik)
                dk = jnp.where(cf, dq, dk)
        if val is None:
            val, idx = dk, ik
        else:
            c = dk < val
            idx = jnp.where(c, ik, idx)
            val = jnp.where(c, dk, val)

    m = jnp.min(val, axis=1, keepdims=True)
    lif = jnp.min(jnp.where(val == m, idx, jnp.inf), axis=1, keepdims=True)
    idx_ref[...] = lif.astype(jnp.int32)
    part = jnp.sum(m)

    @pl.when(r == 0)
    def _init():
        acc_ref[0, 0] = part

    @pl.when(r > 0)
    def _acc():
        acc_ref[0, 0] = acc_ref[0, 0] + part

    @pl.when(r == _RT - 1)
    def _finish():
        loss_ref[...] = jnp.full(
            (1, 1), acc_ref[0, 0] / float(_ROWS * _EMBED), jnp.float32)


def _run_argmin(z3, zn, cn2, cb):
    colf = jnp.arange(_VOCAB, dtype=jnp.float32).reshape(1, _VOCAB)
    return pl.pallas_call(
        _argmin_body,
        grid=(_RT,),
        in_specs=[
            pl.BlockSpec((1, _EMBED, _TM), lambda r: (r // 2, 0, r % 2)),
            pl.BlockSpec((_TM, 1), lambda r: (r, 0)),
            pl.BlockSpec((1, _VOCAB), lambda r: (0, 0)),
            pl.BlockSpec((1, _VOCAB), lambda r: (0, 0)),
            pl.BlockSpec((_VOCAB, _EMBED), lambda r: (0, 0)),
        ],
        out_specs=[
            pl.BlockSpec((_TM, 1), lambda r: (r, 0)),
            pl.BlockSpec((1, 1), lambda r: (0, 0)),
        ],
        out_shape=[
            jax.ShapeDtypeStruct((_ROWS, 1), jnp.int32),
            jax.ShapeDtypeStruct((1, 1), jnp.float32),
        ],
        scratch_shapes=[
            pltpu.SMEM((1, 1), jnp.float32),
        ],
    )(z3, zn, cn2, colf, cb)


def _ppl_body(idx_ref, out_ref):
    idx = idx_ref[...]
    c = jnp.zeros(idx.shape, jnp.int32)
    for b in range(8):
        c = c + (idx == idx[b:b + 1, :]).astype(jnp.int32)
    p = c.astype(jnp.float32) * 0.125
    ent = jnp.sum(jnp.log(p + 1e-10)) * 0.125
    out_ref[...] = jnp.full((1, 1), jnp.exp(-ent), jnp.float32)


def _run_ppl(idx8):
    return pl.pallas_call(
        _ppl_body,
        out_shape=jax.ShapeDtypeStruct((1, 1), jnp.float32),
    )(idx8)


_NC = 2
_NS = 16
_NW = _NC * _NS
_BPW = _ROWS // _NW
_CH = 128
_NCH = _BPW // _CH


@functools.cache
def _make_sc_gather():
    mesh = plsc.VectorSubcoreMesh(core_axis_name="c", subcore_axis_name="s")

    @functools.partial(
        pl.kernel,
        mesh=mesh,
        out_type=jax.ShapeDtypeStruct((_ROWS, _EMBED), jnp.float32),
        scratch_types=[
            pltpu.VMEM((_CH,), jnp.int32),
            pltpu.VMEM((_CH,), jnp.int32),
            pltpu.VMEM((_CH, _EMBED), jnp.float32),
            pltpu.VMEM((_CH, _EMBED), jnp.float32),
            pltpu.SemaphoreType.DMA,
            pltpu.SemaphoreType.DMA,
        ],
    )
    def _sc_gather(table_hbm, idx_hbm, out_hbm, idx0, idx1, rows0, rows1, sem0, sem1):
        wid = lax.axis_index("s") * _NC + lax.axis_index("c")
        base = wid * _BPW
        idx_bufs = (idx0, idx1)
        row_bufs = (rows0, rows1)
        sems = (sem0, sem1)
        copies = []
        for k in range(_NCH):
            pltpu.sync_copy(idx_hbm.at[pl.ds(base + k * _CH, _CH)], idx_bufs[k])
            copies.append(pltpu.async_copy(table_hbm.at[idx_bufs[k]], row_bufs[k], sems[k]))
        for k in range(_NCH):
            copies[k].wait()
            pltpu.sync_copy(row_bufs[k], out_hbm.at[pl.ds(base + k * _CH, _CH)])

    return _sc_gather


def kernel(z, codebook):
    B, C, H, W = z.shape
    z_flat = jnp.transpose(z, (0, 2, 3, 1)).reshape(B, H * W, C).astype(jnp.float32)
    cb = codebook.astype(jnp.float32)
    zn = jnp.sum(z_flat ** 2, axis=-1, keepdims=True)
    cn = jnp.sum(cb ** 2, axis=-1)

    idx2, loss_out = _run_argmin(
        z.reshape(B, C, H * W), zn.reshape(B * H * W, 1),
        cn.reshape(1, _VOCAB), cb)

    idx_flat = idx2.reshape(B * H * W)
    zq_flat = _make_sc_gather()(cb, idx_flat)
    ppl_out = _run_ppl(idx2.reshape(B, H * W))

    z_q = jnp.transpose(zq_flat.reshape(B, H, W, C), (0, 3, 1, 2))
    indices = idx_flat.reshape(B, H, W)
    return z_q, indices, loss_out[0, 0], ppl_out[0, 0]


# speedup vs baseline: 1.0554x; 1.0041x over previous
import functools

import jax
import jax.numpy as jnp
from jax import lax
from jax.experimental import pallas as pl
from jax.experimental.pallas import tpu as pltpu
from jax.experimental.pallas import tpu_sc as plsc

_VOCAB = 8192
_EMBED = 256
_ROWS = 8192
_TM = 512
_RT = _ROWS // _TM
_W = 512


def _argmin_body(z_ref, zn_ref, cn_ref, colf_ref, cb_ref,
                 idx_ref, loss_ref, acc_ref):
    r = pl.program_id(0)
    zt = z_ref[0] * -2.0
    zn = zn_ref[...]

    val = None
    for k in range(_VOCAB // _W):
        off = k * _W
        cbk = cb_ref[off:off + _W, :]
        mmk = lax.dot_general(zt, cbk, (((0,), (1,)), ((), ())),
                              preferred_element_type=jnp.float32)
        cnk = cn_ref[:, pl.ds(off, _W)]
        colk = colf_ref[:, pl.ds(off, _W)]
        dc = (zn + cnk) + mmk
        dk = None
        for q in range(_W // 128):
            dq = dc[:, q * 128:(q + 1) * 128]
            cq = jnp.broadcast_to(colk[:, q * 128:(q + 1) * 128], (_TM, 128))
            if dk is None:
                dk, ik = dq, cq
            else:
                cf = dq < dk
                ik = jnp.where(cf, cq, ik)
                dk = jnp.where(cf, dq, dk)
        if val is None:
            val, idx = dk, ik
        else:
            c = dk < val
            idx = jnp.where(c, ik, idx)
            val = jnp.where(c, dk, val)

    m = jnp.min(val, axis=1, keepdims=True)
    lif = jnp.min(jnp.where(val == m, idx, jnp.inf), axis=1, keepdims=True)
    idx_ref[...] = lif.astype(jnp.int32)
    part = jnp.sum(m)

    @pl.when(r == 0)
    def _init():
        acc_ref[0, 0] = part

    @pl.when(r > 0)
    def _acc():
        acc_ref[0, 0] = acc_ref[0, 0] + part

    @pl.when(r == _RT - 1)
    def _finish():
        loss_ref[...] = jnp.full(
            (1, 1), acc_ref[0, 0] / float(_ROWS * _EMBED), jnp.float32)


def _run_argmin(z3, zn, cn2, cb):
    colf = jnp.arange(_VOCAB, dtype=jnp.float32).reshape(1, _VOCAB)
    return pl.pallas_call(
        _argmin_body,
        grid=(_RT,),
        in_specs=[
            pl.BlockSpec((1, _EMBED, _TM), lambda r: (r // 2, 0, r % 2)),
            pl.BlockSpec((_TM, 1), lambda r: (r, 0)),
            pl.BlockSpec((1, _VOCAB), lambda r: (0, 0)),
            pl.BlockSpec((1, _VOCAB), lambda r: (0, 0)),
            pl.BlockSpec((_VOCAB, _EMBED), lambda r: (0, 0)),
        ],
        out_specs=[
            pl.BlockSpec((_TM, 1), lambda r: (r, 0)),
            pl.BlockSpec((1, 1), lambda r: (0, 0)),
        ],
        out_shape=[
            jax.ShapeDtypeStruct((_ROWS, 1), jnp.int32),
            jax.ShapeDtypeStruct((1, 1), jnp.float32),
        ],
        scratch_shapes=[
            pltpu.SMEM((1, 1), jnp.float32),
        ],
    )(z3, zn, cn2, colf, cb)


def _ppl_body(idx_ref, out_ref):
    idx = idx_ref[...]
    c = jnp.zeros(idx.shape, jnp.int32)
    for b in range(8):
        c = c + (idx == idx[b:b + 1, :]).astype(jnp.int32)
    p = c.astype(jnp.float32) * 0.125
    ent = jnp.sum(jnp.log(p + 1e-10)) * 0.125
    out_ref[...] = jnp.full((1, 1), jnp.exp(-ent), jnp.float32)


def _run_ppl(idx8):
    return pl.pallas_call(
        _ppl_body,
        out_shape=jax.ShapeDtypeStruct((1, 1), jnp.float32),
    )(idx8)


_NC = 2
_NS = 16
_NW = _NC * _NS
_BPW = _ROWS // _NW
_CH = 128
_NCH = _BPW // _CH


@functools.cache
def _make_sc_gather():
    mesh = plsc.VectorSubcoreMesh(core_axis_name="c", subcore_axis_name="s")

    @functools.partial(
        pl.kernel,
        mesh=mesh,
        compiler_params=pltpu.CompilerParams(use_tc_tiling_on_sc=True),
        out_type=jax.ShapeDtypeStruct((_ROWS, _EMBED), jnp.float32),
        scratch_types=[
            pltpu.VMEM((_CH,), jnp.int32),
            pltpu.VMEM((_CH,), jnp.int32),
            pltpu.VMEM((_CH, _EMBED), jnp.float32),
            pltpu.VMEM((_CH, _EMBED), jnp.float32),
            pltpu.SemaphoreType.DMA,
            pltpu.SemaphoreType.DMA,
        ],
    )
    def _sc_gather(table_hbm, idx_hbm, out_hbm, idx0, idx1, rows0, rows1, sem0, sem1):
        wid = lax.axis_index("s") * _NC + lax.axis_index("c")
        base = wid * _BPW
        idx_bufs = (idx0, idx1)
        row_bufs = (rows0, rows1)
        sems = (sem0, sem1)
        copies = []
        for k in range(_NCH):
            pltpu.sync_copy(idx_hbm.at[pl.ds(base + k * _CH, _CH)], idx_bufs[k])
            copies.append(pltpu.async_copy(table_hbm.at[idx_bufs[k]], row_bufs[k], sems[k]))
        for k in range(_NCH):
            copies[k].wait()
            pltpu.sync_copy(row_bufs[k], out_hbm.at[pl.ds(base + k * _CH, _CH)])

    return _sc_gather


def kernel(z, codebook):
    B, C, H, W = z.shape
    z_flat = jnp.transpose(z, (0, 2, 3, 1)).reshape(B, H * W, C).astype(jnp.float32)
    cb = codebook.astype(jnp.float32)
    zn = jnp.sum(z_flat ** 2, axis=-1, keepdims=True)
    cn = jnp.sum(cb ** 2, axis=-1)

    idx2, loss_out = _run_argmin(
        z.reshape(B, C, H * W), zn.reshape(B * H * W, 1),
        cn.reshape(1, _VOCAB), cb)

    idx_flat = idx2.reshape(B * H * W)
    zq_flat = _make_sc_gather()(cb, idx_flat)
    ppl_out = _run_ppl(idx2.reshape(B, H * W))

    z_q = jnp.transpose(zq_flat.reshape(B, H, W, C), (0, 3, 1, 2))
    indices = idx_flat.reshape(B, H, W)
    return z_q, indices, loss_out[0, 0], ppl_out[0, 0]


# speedup vs baseline: 1.0588x; 1.0032x over previous
import functools

import jax
import jax.numpy as jnp
from jax import lax
from jax.experimental import pallas as pl
from jax.experimental.pallas import tpu as pltpu
from jax.experimental.pallas import tpu_sc as plsc

_VOCAB = 8192
_EMBED = 256
_ROWS = 8192
_TM = 512
_RT = _ROWS // _TM
_W = 512


def _argmin_body(z_ref, zn_ref, cn_ref, colf_ref, cb_ref,
                 idx_ref, loss_ref, acc_ref):
    r = pl.program_id(0)
    zt = z_ref[0] * -2.0
    zn = zn_ref[...]

    val = None
    for k in range(_VOCAB // _W):
        off = k * _W
        cbk = cb_ref[off:off + _W, :]
        mmk = lax.dot_general(zt, cbk, (((0,), (1,)), ((), ())),
                              preferred_element_type=jnp.float32)
        cnk = cn_ref[:, pl.ds(off, _W)]
        colk = colf_ref[:, pl.ds(off, _W)]
        dc = (zn + cnk) + mmk
        dk = None
        for q in range(_W // 128):
            dq = dc[:, q * 128:(q + 1) * 128]
            cq = jnp.broadcast_to(colk[:, q * 128:(q + 1) * 128], (_TM, 128))
            if dk is None:
                dk, ik = dq, cq
            else:
                cf = dq < dk
                ik = jnp.where(cf, cq, ik)
                dk = jnp.where(cf, dq, dk)
        if val is None:
            val, idx = dk, ik
        else:
            c = dk < val
            idx = jnp.where(c, ik, idx)
            val = jnp.where(c, dk, val)

    m = jnp.min(val, axis=1, keepdims=True)
    lif = jnp.min(jnp.where(val == m, idx, jnp.inf), axis=1, keepdims=True)
    idx_ref[...] = lif.astype(jnp.int32)
    part = jnp.sum(m)

    @pl.when(r == 0)
    def _init():
        acc_ref[0, 0] = part

    @pl.when(r > 0)
    def _acc():
        acc_ref[0, 0] = acc_ref[0, 0] + part

    @pl.when(r == _RT - 1)
    def _finish():
        loss_ref[...] = jnp.full(
            (1, 1), acc_ref[0, 0] / float(_ROWS * _EMBED), jnp.float32)


def _run_argmin(z3, zn, cn2, cb):
    colf = jnp.arange(_VOCAB, dtype=jnp.float32).reshape(1, _VOCAB)
    return pl.pallas_call(
        _argmin_body,
        grid=(_RT,),
        in_specs=[
            pl.BlockSpec((1, _EMBED, _TM), lambda r: (r // 2, 0, r % 2)),
            pl.BlockSpec((_TM, 1), lambda r: (r, 0)),
            pl.BlockSpec((1, _VOCAB), lambda r: (0, 0)),
            pl.BlockSpec((1, _VOCAB), lambda r: (0, 0)),
            pl.BlockSpec((_VOCAB, _EMBED), lambda r: (0, 0)),
        ],
        out_specs=[
            pl.BlockSpec((_TM, 1), lambda r: (r, 0)),
            pl.BlockSpec((1, 1), lambda r: (0, 0)),
        ],
        out_shape=[
            jax.ShapeDtypeStruct((_ROWS, 1), jnp.int32),
            jax.ShapeDtypeStruct((1, 1), jnp.float32),
        ],
        scratch_shapes=[
            pltpu.SMEM((1, 1), jnp.float32),
        ],
    )(z3, zn, cn2, colf, cb)


def _ppl_body(idx_ref, out_ref):
    idx = idx_ref[...]
    c = jnp.zeros(idx.shape, jnp.int32)
    for b in range(8):
        c = c + (idx == idx[b:b + 1, :]).astype(jnp.int32)
    p = c.astype(jnp.float32) * 0.125
    ent = jnp.sum(jnp.log(p + 1e-10)) * 0.125
    out_ref[...] = jnp.full((1, 1), jnp.exp(-ent), jnp.float32)


def _run_ppl(idx8):
    return pl.pallas_call(
        _ppl_body,
        out_shape=jax.ShapeDtypeStruct((1, 1), jnp.float32),
    )(idx8)


_NC = 2
_NS = 16
_NW = _NC * _NS
_BPW = _ROWS // _NW
_CH = 128
_NCH = _BPW // _CH


@functools.cache
def _make_sc_gather():
    mesh = plsc.VectorSubcoreMesh(core_axis_name="c", subcore_axis_name="s")

    @functools.partial(
        pl.kernel,
        mesh=mesh,
        out_type=jax.ShapeDtypeStruct((_ROWS, _EMBED), jnp.float32),
        scratch_types=[
            pltpu.VMEM((_CH,), jnp.int32),
            pltpu.VMEM((_CH,), jnp.int32),
            pltpu.VMEM((_CH, _EMBED), jnp.float32),
            pltpu.VMEM((_CH, _EMBED), jnp.float32),
            pltpu.SemaphoreType.DMA,
            pltpu.SemaphoreType.DMA,
        ],
    )
    def _sc_gather(table_hbm, idx_hbm, out_hbm, idx0, idx1, rows0, rows1, sem0, sem1):
        wid = lax.axis_index("s") * _NC + lax.axis_index("c")
        base = wid * _BPW
        idx_bufs = (idx0, idx1)
        row_bufs = (rows0, rows1)
        sems = (sem0, sem1)
        copies = []
        for k in range(_NCH):
            pltpu.sync_copy(idx_hbm.at[pl.ds(base + k * _CH, _CH)], idx_bufs[k])
            copies.append(pltpu.async_copy(table_hbm.at[idx_bufs[k]], row_bufs[k], sems[k]))
        for k in range(_NCH):
            copies[k].wait()
            pltpu.sync_copy(row_bufs[k], out_hbm.at[pl.ds(base + k * _CH, _CH)])

    return _sc_gather


def kernel(z, codebook):
    B, C, H, W = z.shape
    z_flat = jnp.transpose(z, (0, 2, 3, 1)).reshape(B, H * W, C).astype(jnp.float32)
    cb = codebook.astype(jnp.float32)
    zn = jnp.sum(z_flat ** 2, axis=-1, keepdims=True)
    cn = jnp.sum(cb ** 2, axis=-1)

    idx2, loss_out = _run_argmin(
        z.reshape(B, C, H * W), zn.reshape(B * H * W, 1),
        cn.reshape(1, _VOCAB), cb)

    idx_flat = idx2.reshape(B * H * W)
    zq_flat = _make_sc_gather()(cb, idx_flat)
    ppl_out = _run_ppl(idx2.reshape(B, H * W))

    z_q = jnp.transpose(zq_flat.reshape(B, H, W, C), (0, 3, 1, 2))
    indices = idx_flat.reshape(B, H, W)
    return z_q, indices, loss_out[0, 0], ppl_out[0, 0]
